# baseline (speedup 1.0000x reference)
import functools

import jax
import jax.numpy as jnp
from jax import lax
from jax.experimental import pallas as pl
from jax.experimental.pallas import tpu as pltpu
from jax.experimental.pallas import tpu_sc as plsc

SCALE = 8.0
NW = 32
C = 128
LANES = 16


def kernel(input, table):
    B0, B1 = input.shape
    B = B0 * B1
    V, D = table.shape

    BW = B // NW
    NCHUNK = BW // C

    idx = input.reshape(NW, NCHUNK, C)

    mesh = plsc.VectorSubcoreMesh(core_axis_name="c", subcore_axis_name="s")

    @functools.partial(
        pl.kernel,
        mesh=mesh,
        out_type=jax.ShapeDtypeStruct((B, D), jnp.float32),
        scratch_types=[
            pltpu.VMEM((NCHUNK, C), jnp.int32),
            pltpu.VMEM((C, D), jnp.float32),
            pltpu.SemaphoreType.DMA,
        ],
        compiler_params=pltpu.CompilerParams(use_tc_tiling_on_sc=False),
    )
    def emb(idx_hbm, table_hbm, out_hbm, idx_v, rows_v, sem):
        wid = lax.axis_index("s") * 2 + lax.axis_index("c")
        base = wid * BW
        pltpu.sync_copy(idx_hbm.at[wid], idx_v)

        def chunk_body(c, carry):
            pltpu.async_copy(table_hbm.at[idx_v.at[c]], rows_v, sem).wait()

            def row_body(r, carry2):
                for s in range(D // LANES):
                    sl = pl.ds(s * LANES, LANES)
                    rows_v[r, sl] = rows_v[r, sl] * SCALE
                return carry2

            lax.fori_loop(0, C, row_body, 0)
            pltpu.sync_copy(rows_v, out_hbm.at[pl.ds(base + c * C, C)])
            return carry

        lax.fori_loop(0, NCHUNK, chunk_body, 0)

    out = emb(idx, table)
    return out.reshape(B0, B1, D)


# speedup vs baseline: 1.2073x; 1.2073x over previous
import functools

import jax
import jax.numpy as jnp
from jax import lax
from jax.experimental import pallas as pl
from jax.experimental.pallas import tpu as pltpu
from jax.experimental.pallas import tpu_sc as plsc

SCALE = 8.0
NW = 32
LANES = 16
NBUF = 4
CA, CB = 128, 72


def kernel(input, table):
    R, S = input.shape
    V, D = table.shape
    RW = R // NW
    row_bytes = S * D * 4

    mesh = plsc.VectorSubcoreMesh(core_axis_name="c", subcore_axis_name="s")

    @functools.partial(
        pl.kernel,
        mesh=mesh,
        out_type=jax.ShapeDtypeStruct((R, S, D), jnp.float32),
        scratch_types=[
            pltpu.VMEM((RW, CA), jnp.int32),
            pltpu.VMEM((RW, CB), jnp.int32),
            [pltpu.VMEM((S, D), jnp.float32) for _ in range(NBUF)],
            [pltpu.SemaphoreType.DMA for _ in range(NBUF)],
            [pltpu.SemaphoreType.DMA for _ in range(NBUF)],
        ],
        compiler_params=pltpu.CompilerParams(use_tc_tiling_on_sc=False),
    )
    def emb(idx_hbm, table_hbm, out_hbm, idx_a, idx_b, bufs, gsems, ssems):
        wid = lax.axis_index("s") * 2 + lax.axis_index("c")
        base = wid * RW
        pltpu.sync_copy(idx_hbm.at[pl.ds(base, RW), pl.ds(0, CA)], idx_a)
        pltpu.sync_copy(idx_hbm.at[pl.ds(base, RW), pl.ds(CA, CB)], idx_b)

        def fire(r, t):
            pltpu.async_copy(
                table_hbm.at[idx_a.at[r]], bufs[t].at[pl.ds(0, CA)], gsems[t])
            pltpu.async_copy(
                table_hbm.at[idx_b.at[r]], bufs[t].at[pl.ds(CA, CB)], gsems[t])

        def drain(sem, t):
            pltpu.make_async_copy(
                table_hbm.at[pl.ds(0, S)], bufs[t], sem).wait()

        def scale(t):
            def row_body(r2, carry):
                for s in range(D // LANES):
                    sl = pl.ds(s * LANES, LANES)
                    bufs[t][r2, sl] = bufs[t][r2, sl] * SCALE
                return carry
            lax.fori_loop(0, S, row_body, 0)

        for t in range(NBUF - 1):
            fire(t, t)

        def body(i, carry):
            for t in range(NBUF):
                r = i * NBUF + t
                drain(gsems[t], t)
                scale(t)
                pltpu.async_copy(bufs[t], out_hbm.at[base + r], ssems[t])
                nt = (t + NBUF - 1) % NBUF
                nr = r + NBUF - 1

                @pl.when(jnp.logical_and(r >= 1, nr <= RW - 1))
                def _():
                    drain(ssems[nt], nt)

                @pl.when(nr <= RW - 1)
                def _():
                    fire(nr, nt)
            return carry

        lax.fori_loop(0, RW // NBUF, body, 0)
        for t in range(NBUF):
            drain(ssems[t], t)

    return emb(input, table)
